# baseline (speedup 1.0000x reference)
import numpy as np
import jax
import jax.numpy as jnp
from jax.experimental import pallas as pl
from jax.experimental.pallas import tpu as pltpu

P_ = 8
OBS_ = 8
TS_ = 4
DH_ = 64
N_ = 64
BB_ = 16


def _center_haar_matrix() -> np.ndarray:
    C = np.eye(16, dtype=np.float64)
    for c in range(2):
        for f in range(8):
            C[14 + c, f * 2 + c] -= 1.0
    s = float(np.float32(np.sqrt(2.0)))
    H = np.zeros((16, 16), dtype=np.float64)
    for t in range(4):
        for c in range(2):
            H[(2 * t) * 2 + c, t * 4 + c] = 1.0 / s
            H[(2 * t + 1) * 2 + c, t * 4 + c] = 1.0 / s
            H[(2 * t) * 2 + c, t * 4 + (c + 2)] = 1.0 / s
            H[(2 * t + 1) * 2 + c, t * 4 + (c + 2)] = -1.0 / s
    return (C @ H).astype(np.float32).reshape(16, 4, 4)


_M3 = _center_haar_matrix()


def _rl_kernel(xe_ref, xn_ref, sel_ref, A_ref, bte_ref, W1_ref, b1_ref,
               W2_ref, b2_ref, Wce_ref, bce_ref, fre_ref, rem_ref):
    BB = xe_ref.shape[0]
    R = BB * N_
    bf16 = jnp.bfloat16
    f32 = jnp.float32

    Ab = A_ref[...].astype(bf16)
    bte = bte_ref[...]
    W1 = W1_ref[...].astype(bf16)
    W2 = W2_ref[...].astype(bf16)
    b1 = b1_ref[...]
    b2 = b2_ref[...]
    Wce = Wce_ref[...]
    bce = bce_ref[...]
    xn = xn_ref[...]
    xg = jax.lax.dot_general(
        xn, sel_ref[...], (((1,), (0,)), ((), ())),
        precision=jax.lax.Precision.HIGHEST, preferred_element_type=f32)

    fe = jnp.maximum(
        jnp.dot(xe_ref[...].astype(bf16), Ab, preferred_element_type=f32) + bte, 0.0)
    fn = jnp.maximum(
        jnp.dot(xn.astype(bf16), Ab, preferred_element_type=f32) + bte, 0.0)
    f = (fn.reshape(BB, N_, TS_ * DH_) * fe[:, None, :]).reshape(R, TS_ * DH_)

    two_pi = 2.0 * np.pi
    c0 = xg[:, 0:TS_]
    c1 = xg[:, TS_:2 * TS_]
    dist = jnp.sqrt(c0 * c0 + c1 * c1)
    ang = jnp.arctan2(c0, c1)
    ang = jnp.where(ang < 0.0, ang + two_pi, ang)
    pid = (ang / (2.0 * np.pi / P_)).astype(jnp.int32)
    nonself = (xn[:, 14:15] + xn[:, 15:16]) != 0.0
    ok = jnp.logical_and((c0 + c1) != 0.0, nonself)
    pid = jnp.where(ok, pid, -1)

    rowb = jax.lax.broadcasted_iota(jnp.int32, (R, 1), 0) // N_
    colio = jax.lax.broadcasted_iota(jnp.int32, (R, P_ * BB), 1)
    ones_col = jnp.ones((R, 1), f32)

    for t in range(TS_):
        ft = f[:, t * DH_:(t + 1) * DH_]
        h = jnp.maximum(
            jnp.dot(ft.astype(bf16), W1, preferred_element_type=f32) + b1, 0.0)
        frt = jnp.maximum(
            jnp.dot(h.astype(bf16), W2, preferred_element_type=f32) + b2, 0.0)
        fre_ref[:, t, :] = frt

        pid_t = pid[:, t:t + 1]
        col = jnp.where(pid_t >= 0, rowb * P_ + pid_t, -1)
        onehot = colio == col
        G = jnp.concatenate(
            [frt, dist[:, t:t + 1], ang[:, t:t + 1], ones_col], axis=1)
        S = jax.lax.dot_general(
            onehot.astype(bf16), G.astype(bf16),
            (((0,), (0,)), ((), ())), preferred_element_type=f32)

        n = S[:, DH_ + 2:DH_ + 3] + 0.0001
        rp = S[:, :DH_] / n
        pos_d = S[:, DH_:DH_ + 1] / n
        pos_a = S[:, DH_ + 1:DH_ + 2] / n
        fp = jnp.maximum(pos_d * Wce[0:1, :] + pos_a * Wce[1:2, :] + bce, 0.0)
        row = jnp.concatenate([rp, fp], axis=1)
        rem_ref[:, t * P_:(t + 1) * P_, :] = row.reshape(BB, P_, 2 * DH_)


@jax.jit
def kernel(x_ego_2d, x_nei_2d, W_te, b_te, W_fc1, b_fc1, W_fc2, b_fc2, W_ce, b_ce):
    B = x_ego_2d.shape[0]
    f32 = jnp.float32
    BN = B * N_

    xe = x_ego_2d.reshape(B, OBS_ * 2)
    xn = x_nei_2d.reshape(BN, OBS_ * 2)
    sel = np.zeros((16, 8), dtype=np.float32)
    for t in range(TS_):
        sel[4 * t, t] = 1.0
        sel[4 * t + 1, TS_ + t] = 1.0
    sel = jnp.asarray(sel)
    A = jnp.einsum('itc,cd->itd', jnp.asarray(_M3), W_te).reshape(16, TS_ * DH_)
    bte = jnp.tile(b_te, TS_).reshape(1, TS_ * DH_)

    grid = (B // BB_,)
    full = lambda shape: pl.BlockSpec(shape, lambda i: (0,) * len(shape))
    fre, rem = pl.pallas_call(
        _rl_kernel,
        grid=grid,
        in_specs=[
            pl.BlockSpec((BB_, 16), lambda i: (i, 0)),
            pl.BlockSpec((BB_ * N_, 16), lambda i: (i, 0)),
            full((16, 2 * TS_)),
            full((16, TS_ * DH_)),
            full((1, TS_ * DH_)),
            full((DH_, DH_)),
            full((1, DH_)),
            full((DH_, DH_)),
            full((1, DH_)),
            full((2, DH_)),
            full((1, DH_)),
        ],
        out_specs=[
            pl.BlockSpec((BB_ * N_, TS_, DH_), lambda i: (i, 0, 0)),
            pl.BlockSpec((BB_, TS_ * P_, 2 * DH_), lambda i: (i, 0, 0)),
        ],
        out_shape=[
            jax.ShapeDtypeStruct((BN, TS_, DH_), f32),
            jax.ShapeDtypeStruct((B, TS_ * P_, 2 * DH_), f32),
        ],
        compiler_params=pltpu.CompilerParams(
            dimension_semantics=("arbitrary",)),
    )(xe, xn, sel, A, bte, W_fc1, b_fc1.reshape(1, DH_), W_fc2,
      b_fc2.reshape(1, DH_), W_ce, b_ce.reshape(1, DH_))

    re_matrix = rem.reshape(B, TS_, P_, 2 * DH_)
    f_re = fre.reshape(B, N_, TS_, DH_)
    return (re_matrix, f_re)


# speedup vs baseline: 1.0832x; 1.0832x over previous
import numpy as np
import jax
import jax.numpy as jnp
from jax.experimental import pallas as pl
from jax.experimental.pallas import tpu as pltpu

P_ = 8
OBS_ = 8
TS_ = 4
DH_ = 64
N_ = 64
BB_ = 8


def _center_haar_matrix() -> np.ndarray:
    C = np.eye(16, dtype=np.float64)
    for c in range(2):
        for f in range(8):
            C[14 + c, f * 2 + c] -= 1.0
    s = float(np.float32(np.sqrt(2.0)))
    H = np.zeros((16, 16), dtype=np.float64)
    for t in range(4):
        for c in range(2):
            H[(2 * t) * 2 + c, t * 4 + c] = 1.0 / s
            H[(2 * t + 1) * 2 + c, t * 4 + c] = 1.0 / s
            H[(2 * t) * 2 + c, t * 4 + (c + 2)] = 1.0 / s
            H[(2 * t + 1) * 2 + c, t * 4 + (c + 2)] = -1.0 / s
    return (C @ H).astype(np.float32).reshape(16, 4, 4)


_M3 = _center_haar_matrix()


def _rl_kernel(xe_ref, xn_ref, xg_ref, A_ref, bte_ref, W1_ref, b1_ref,
               W2_ref, b2_ref, Wce_ref, bce_ref, fre_ref, rem_ref):
    BB = xe_ref.shape[0]
    R = BB * N_
    bf16 = jnp.bfloat16
    f32 = jnp.float32

    Ab = A_ref[...].astype(bf16)
    bte = bte_ref[...]
    W1 = W1_ref[...].astype(bf16)
    W2 = W2_ref[...].astype(bf16)
    b1 = b1_ref[...]
    b2 = b2_ref[...]
    Wce = Wce_ref[...]
    bce = bce_ref[...]
    xn = xn_ref[...]
    xg = xg_ref[...]

    fe = jnp.maximum(
        jnp.dot(xe_ref[...].astype(bf16), Ab, preferred_element_type=f32) + bte, 0.0)
    fn = jnp.maximum(
        jnp.dot(xn.astype(bf16), Ab, preferred_element_type=f32) + bte, 0.0)
    f = (fn.reshape(BB, N_, TS_ * DH_) * fe[:, None, :]).reshape(R, TS_ * DH_)

    two_pi = 2.0 * np.pi
    c0 = xg[:, 0:TS_]
    c1 = xg[:, TS_:2 * TS_]
    dist = jnp.sqrt(c0 * c0 + c1 * c1)
    ang = jnp.arctan2(c0, c1)
    ang = jnp.where(ang < 0.0, ang + two_pi, ang)
    pid = (ang / (2.0 * np.pi / P_)).astype(jnp.int32)
    nonself = (xn[:, 14:15] + xn[:, 15:16]) != 0.0
    ok = jnp.logical_and((c0 + c1) != 0.0, nonself)
    pid = jnp.where(ok, pid, -1)

    rowb = jax.lax.broadcasted_iota(jnp.int32, (R, 1), 0) // N_
    colio = jax.lax.broadcasted_iota(jnp.int32, (R, P_ * BB), 1)
    ones_col = jnp.ones((R, 1), f32)

    for t in range(TS_):
        ft = f[:, t * DH_:(t + 1) * DH_]
        h = jnp.maximum(
            jnp.dot(ft.astype(bf16), W1, preferred_element_type=f32) + b1, 0.0)
        frt = jnp.maximum(
            jnp.dot(h.astype(bf16), W2, preferred_element_type=f32) + b2, 0.0)
        fre_ref[:, t, :] = frt

        pid_t = pid[:, t:t + 1]
        col = jnp.where(pid_t >= 0, rowb * P_ + pid_t, -1)
        onehot = colio == col
        G = jnp.concatenate(
            [frt, dist[:, t:t + 1], ang[:, t:t + 1], ones_col], axis=1)
        S = jax.lax.dot_general(
            onehot.astype(bf16), G.astype(bf16),
            (((0,), (0,)), ((), ())), preferred_element_type=f32)

        n = S[:, DH_ + 2:DH_ + 3] + 0.0001
        rp = S[:, :DH_] / n
        pos_d = S[:, DH_:DH_ + 1] / n
        pos_a = S[:, DH_ + 1:DH_ + 2] / n
        fp = jnp.maximum(pos_d * Wce[0:1, :] + pos_a * Wce[1:2, :] + bce, 0.0)
        row = jnp.concatenate([rp, fp], axis=1)
        rem_ref[:, t * P_:(t + 1) * P_, :] = row.reshape(BB, P_, 2 * DH_)


@jax.jit
def kernel(x_ego_2d, x_nei_2d, W_te, b_te, W_fc1, b_fc1, W_fc2, b_fc2, W_ce, b_ce):
    B = x_ego_2d.shape[0]
    f32 = jnp.float32
    BN = B * N_

    xe = x_ego_2d.reshape(B, OBS_ * 2)
    xn = x_nei_2d.reshape(BN, OBS_ * 2)
    xg = jnp.concatenate(
        [x_nei_2d[:, :, ::2, 0], x_nei_2d[:, :, ::2, 1]], axis=-1
    ).reshape(BN, 2 * TS_)
    A = jnp.einsum('itc,cd->itd', jnp.asarray(_M3), W_te).reshape(16, TS_ * DH_)
    bte = jnp.tile(b_te, TS_).reshape(1, TS_ * DH_)

    grid = (B // BB_,)
    full = lambda shape: pl.BlockSpec(shape, lambda i: (0,) * len(shape))
    fre, rem = pl.pallas_call(
        _rl_kernel,
        grid=grid,
        in_specs=[
            pl.BlockSpec((BB_, 16), lambda i: (i, 0)),
            pl.BlockSpec((BB_ * N_, 16), lambda i: (i, 0)),
            pl.BlockSpec((BB_ * N_, 2 * TS_), lambda i: (i, 0)),
            full((16, TS_ * DH_)),
            full((1, TS_ * DH_)),
            full((DH_, DH_)),
            full((1, DH_)),
            full((DH_, DH_)),
            full((1, DH_)),
            full((2, DH_)),
            full((1, DH_)),
        ],
        out_specs=[
            pl.BlockSpec((BB_ * N_, TS_, DH_), lambda i: (i, 0, 0)),
            pl.BlockSpec((BB_, TS_ * P_, 2 * DH_), lambda i: (i, 0, 0)),
        ],
        out_shape=[
            jax.ShapeDtypeStruct((BN, TS_, DH_), f32),
            jax.ShapeDtypeStruct((B, TS_ * P_, 2 * DH_), f32),
        ],
        compiler_params=pltpu.CompilerParams(
            dimension_semantics=("arbitrary",)),
    )(xe, xn, xg, A, bte, W_fc1, b_fc1.reshape(1, DH_), W_fc2,
      b_fc2.reshape(1, DH_), W_ce, b_ce.reshape(1, DH_))

    re_matrix = rem.reshape(B, TS_, P_, 2 * DH_)
    f_re = fre.reshape(B, N_, TS_, DH_)
    return (re_matrix, f_re)
